# baseline (speedup 1.0000x reference)
import functools

import jax
import jax.numpy as jnp
from jax import lax
from jax.experimental import pallas as pl
from jax.experimental.pallas import tpu as pltpu
from jax.experimental.pallas import tpu_sc as plsc

F32 = jnp.float32

NC = 2
NS = 16
NW = NC * NS


def _pq_body(x_ref, we1_ref, wed_ref, out_ref):
    wc = jnp.dot(we1_ref[0:256, :], wed_ref[...], preferred_element_type=F32)
    w = jnp.concatenate([wc[0:128, :], wc[128:256, :]], axis=1)
    out_ref[...] = jnp.dot(x_ref[...], w, preferred_element_type=F32)


def _pq_call(x, We1, Wed):
    n, dn = x.shape
    blk = 1000
    grid = n // blk
    return pl.pallas_call(
        _pq_body,
        grid=(grid,),
        in_specs=[
            pl.BlockSpec((blk, dn), lambda i: (i, 0)),
            pl.BlockSpec(We1.shape, lambda i: (0, 0)),
            pl.BlockSpec(Wed.shape, lambda i: (0, 0)),
        ],
        out_specs=pl.BlockSpec((blk, 32), lambda i: (i, 0)),
        out_shape=jax.ShapeDtypeStruct((n, 32), F32),
    )(x, We1, Wed)


def _cef_body(ef_ref, we1_ref, wed_ref, be1_ref, bed_ref, g_ref, out_ref):
    wed = wed_ref[...]
    wce = jnp.dot(we1_ref[256:272, :], wed, preferred_element_type=F32)
    wcg = jnp.dot(we1_ref[272:304, :], wed, preferred_element_type=F32)
    ct = (jnp.dot(g_ref[...], wcg, preferred_element_type=F32)
          + jnp.dot(be1_ref[...], wed, preferred_element_type=F32)
          + bed_ref[...])
    e = ef_ref[...]
    out_ref[0, :, :] = jnp.dot(e, wce, preferred_element_type=F32) + ct
    out_ref[1, :, :] = e


def _cef_call(ef, We1, Wed, be1, bed, g):
    e_num, de = ef.shape
    blk = 8000
    grid = e_num // blk
    return pl.pallas_call(
        _cef_body,
        grid=(grid,),
        in_specs=[
            pl.BlockSpec((blk, de), lambda i: (i, 0)),
            pl.BlockSpec(We1.shape, lambda i: (0, 0)),
            pl.BlockSpec(Wed.shape, lambda i: (0, 0)),
            pl.BlockSpec(be1.shape, lambda i: (0, 0)),
            pl.BlockSpec(bed.shape, lambda i: (0, 0)),
            pl.BlockSpec(g.shape, lambda i: (0, 0)),
        ],
        out_specs=pl.BlockSpec((2, blk, de), lambda i: (0, i, 0)),
        out_shape=jax.ShapeDtypeStruct((2, e_num, de), F32),
    )(ef, We1, Wed, be1, bed, g)


def _sc_edge_body(n, e_num, k, nr, pq_hbm, cef_hbm, src_hbm, dst_hbm, z_hbm,
                  efo_hbm, acc_hbm,
                  idx_s, idx_d, sidx_s, sidx_d, gsrc, gdst, ceb, efb,
                  payf, payb, eoutb, acc, pqs,
                  s_is, s_id, s_gs, s_gd, s_ce, s_ef, s_eo, s_pf, s_pb):
    cid = lax.axis_index("c")
    sid = lax.axis_index("s")
    wid = sid * NC + cid
    nchunks = e_num // k
    nch = jnp.int32(nchunks // NW) + (wid < jnp.int32(nchunks % NW))
    stripe = 1000
    n_stripes = n // stripe

    @pl.when(sid < n_stripes)
    def _():
        pltpu.sync_copy(z_hbm.at[pl.ds(sid * stripe, stripe)],
                        acc.at[pl.ds(sid * stripe, stripe)])
        pltpu.sync_copy(pq_hbm.at[pl.ds(sid * stripe, stripe)],
                        pqs.at[pl.ds(sid * stripe, stripe)])

    cnt_row = jnp.where(lax.iota(jnp.int32, 16) == 0,
                        jnp.float32(1.0), jnp.float32(0.0))
    for b in range(2):
        for e in range(k):
            payf[b][e, 16:32] = cnt_row
            payb[b][e, 16:32] = cnt_row

    plsc.subcore_barrier()

    def off(r):
        return (r * NW + wid) * k

    def issue_idx(r, b):
        @pl.when(r < nch)
        def _():
            pltpu.async_copy(src_hbm.at[pl.ds(off(r), k)], idx_s[b], s_is[b])
            pltpu.async_copy(dst_hbm.at[pl.ds(off(r), k)], idx_d[b], s_id[b])

    def wait_idx(r, b):
        @pl.when(r < nch)
        def _():
            pltpu.make_async_copy(src_hbm.at[pl.ds(0, k)], idx_s[b],
                                  s_is[b]).wait()
            pltpu.make_async_copy(dst_hbm.at[pl.ds(0, k)], idx_d[b],
                                  s_id[b]).wait()

    def issue_gather(r, b):
        @pl.when(r < nch)
        def _():
            pltpu.async_copy(pqs.at[idx_s[b]], gsrc[b], s_gs[b])
            pltpu.async_copy(pqs.at[idx_d[b]], gdst[b], s_gd[b])
            pltpu.async_copy(cef_hbm.at[0].at[pl.ds(off(r), k)], ceb[b],
                             s_ce[b])
            pltpu.async_copy(cef_hbm.at[1].at[pl.ds(off(r), k)], efb[b],
                             s_ef[b])

    def wait_gather(r, b):
        @pl.when(r < nch)
        def _():
            pltpu.make_async_copy(pqs.at[idx_s[b]], gsrc[b], s_gs[b]).wait()
            pltpu.make_async_copy(pqs.at[idx_d[b]], gdst[b], s_gd[b]).wait()
            pltpu.make_async_copy(cef_hbm.at[0].at[pl.ds(0, k)], ceb[b],
                                  s_ce[b]).wait()
            pltpu.make_async_copy(cef_hbm.at[1].at[pl.ds(0, k)], efb[b],
                                  s_ef[b]).wait()

    def issue_store(r, b):
        @pl.when(r < nch)
        def _():
            pltpu.async_copy(eoutb[b], efo_hbm.at[pl.ds(off(r), k)], s_eo[b])
            pltpu.async_copy(payf[b], acc.at[sidx_d[b]], s_pf[b], add=True)
            pltpu.async_copy(payb[b], acc.at[sidx_s[b]], s_pb[b], add=True)

    def wait_store(r, b):
        @pl.when((r >= 0) & (r < nch))
        def _():
            pltpu.make_async_copy(eoutb[b], efo_hbm.at[pl.ds(0, k)],
                                  s_eo[b]).wait()
            pltpu.make_async_copy(payf[b], acc.at[sidx_d[b]], s_pf[b]).wait()
            pltpu.make_async_copy(payb[b], acc.at[sidx_s[b]], s_pb[b]).wait()

    issue_idx(0, 0)
    issue_idx(1, 1)
    wait_idx(0, 0)
    issue_gather(0, 0)

    @pl.loop(0, nr, step=2)
    def _round_pair(i):
        for db in range(2):
            j = i + db
            b = db
            wait_store(j - 2, b)
            wait_gather(j, b)

            @pl.when(j < nch)
            def _():
                for t in range(k // 16):
                    sidx_s[b][pl.ds(t * 16, 16)] = idx_s[b][pl.ds(t * 16, 16)]
                    sidx_d[b][pl.ds(t * 16, 16)] = idx_d[b][pl.ds(t * 16, 16)]

            wait_idx(j + 1, 1 - b)
            issue_gather(j + 1, 1 - b)
            issue_idx(j + 2, b)

            @pl.when(j < nch)
            def _():
                for e in range(k):
                    ps = gsrc[b][e, 0:16]
                    qs = gsrc[b][e, 16:32]
                    pd = gdst[b][e, 0:16]
                    qd = gdst[b][e, 16:32]
                    ce = ceb[b][e, 0:16]
                    efv = efb[b][e, 0:16]
                    fwd = ps + qd + ce
                    bwd = pd + qs + ce
                    payf[b][e, 0:16] = fwd
                    payb[b][e, 0:16] = bwd
                    eoutb[b][e, 0:16] = (fwd + bwd) * jnp.float32(0.5) + efv

            issue_store(j, b)

    wait_store(nr - 2, 0)
    wait_store(nr - 1, 1)

    plsc.subcore_barrier()

    @pl.when(sid < n_stripes)
    def _():
        pltpu.sync_copy(acc.at[pl.ds(sid * stripe, stripe)],
                        acc_hbm.at[cid].at[pl.ds(sid * stripe, stripe)])


def _sc_edge_call(pq, cef, src, dst, zeros):
    n = pq.shape[0]
    e_num = src.shape[0]
    k = 128
    nr = -(-(e_num // k) // NW)
    nr += nr % 2
    mesh = plsc.VectorSubcoreMesh(core_axis_name="c", subcore_axis_name="s")
    f = pl.kernel(
        functools.partial(_sc_edge_body, n, e_num, k, nr),
        out_type=[
            jax.ShapeDtypeStruct((e_num, 16), F32),
            jax.ShapeDtypeStruct((NC, n, 32), F32),
        ],
        mesh=mesh,
        scratch_types=[
            [pltpu.VMEM((k,), jnp.int32) for _ in range(2)],
            [pltpu.VMEM((k,), jnp.int32) for _ in range(2)],
            [pltpu.VMEM((k,), jnp.int32) for _ in range(2)],
            [pltpu.VMEM((k,), jnp.int32) for _ in range(2)],
            [pltpu.VMEM((k, 32), F32) for _ in range(2)],
            [pltpu.VMEM((k, 32), F32) for _ in range(2)],
            [pltpu.VMEM((k, 16), F32) for _ in range(2)],
            [pltpu.VMEM((k, 16), F32) for _ in range(2)],
            [pltpu.VMEM((k, 32), F32) for _ in range(2)],
            [pltpu.VMEM((k, 32), F32) for _ in range(2)],
            [pltpu.VMEM((k, 16), F32) for _ in range(2)],
            pltpu.VMEM_SHARED((n, 32), F32),
            pltpu.VMEM_SHARED((n, 32), F32),
            [pltpu.SemaphoreType.DMA for _ in range(2)],
            [pltpu.SemaphoreType.DMA for _ in range(2)],
            [pltpu.SemaphoreType.DMA for _ in range(2)],
            [pltpu.SemaphoreType.DMA for _ in range(2)],
            [pltpu.SemaphoreType.DMA for _ in range(2)],
            [pltpu.SemaphoreType.DMA for _ in range(2)],
            [pltpu.SemaphoreType.DMA for _ in range(2)],
            [pltpu.SemaphoreType.DMA for _ in range(2)],
            [pltpu.SemaphoreType.DMA for _ in range(2)],
        ],
        compiler_params=pltpu.CompilerParams(use_tc_tiling_on_sc=False),
    )
    return f(pq, cef, src, dst, zeros)


def _node_body(nblk, n, e_num, x_ref, acc_ref, g_ref, wn1_ref, bn1_ref,
               wnd_ref, bnd_ref, wg1_ref, bg1_ref, wgd_ref, bgd_ref,
               nf_ref, gf_ref, snf_ref, seg_ref):
    i = pl.program_id(0)
    acc0 = acc_ref[0]
    acc1 = acc_ref[1]
    data = acc0[:, 0:16] + acc1[:, 0:16]
    cnt = acc0[:, 16:17] + acc1[:, 16:17]
    em = data / jnp.maximum(cnt, 1.0)

    wnd = wnd_ref[...]
    wn = jnp.dot(wn1_ref[...], wnd, preferred_element_type=F32)
    gterm = (jnp.dot(g_ref[...], wn[144:176, :], preferred_element_type=F32)
             + jnp.dot(bn1_ref[...], wnd, preferred_element_type=F32)
             + bnd_ref[...])
    xb = x_ref[...]
    nf_new = (jnp.dot(xb, wn[0:128, :], preferred_element_type=F32)
              + jnp.dot(em, wn[128:144, :], preferred_element_type=F32)
              + gterm)
    nf_ref[...] = nf_new + xb

    part_nf = jnp.sum(nf_new, axis=0, keepdims=True)
    part_eg = jnp.sum(data, axis=0, keepdims=True)

    @pl.when(i == 0)
    def _():
        snf_ref[...] = part_nf
        seg_ref[...] = part_eg

    @pl.when(i > 0)
    def _():
        snf_ref[...] += part_nf
        seg_ref[...] += part_eg

    @pl.when(i == nblk - 1)
    def _():
        ng = snf_ref[...] * jnp.float32(1.0 / n)
        eg = seg_ref[...] * jnp.float32(1.0 / (2.0 * e_num))
        wgd = wgd_ref[...]
        wg = jnp.dot(wg1_ref[...], wgd, preferred_element_type=F32)
        gf_new = (jnp.dot(eg, wg[0:16, :], preferred_element_type=F32)
                  + jnp.dot(ng, wg[16:144, :], preferred_element_type=F32)
                  + jnp.dot(g_ref[...], wg[144:176, :],
                            preferred_element_type=F32)
                  + jnp.dot(bg1_ref[...], wgd, preferred_element_type=F32)
                  + bgd_ref[...])
        gf_ref[...] = gf_new + g_ref[...]


def _node_call(x, acc, g, Wn1, bn1, Wnd, bnd, Wg1, bg1, Wgd, bgd, e_num):
    n, dn = x.shape
    blk = 1000
    grid = n // blk
    return pl.pallas_call(
        functools.partial(_node_body, grid, n, e_num),
        grid=(grid,),
        in_specs=[
            pl.BlockSpec((blk, dn), lambda i: (i, 0)),
            pl.BlockSpec((2, blk, 32), lambda i: (0, i, 0)),
            pl.BlockSpec(g.shape, lambda i: (0, 0)),
            pl.BlockSpec(Wn1.shape, lambda i: (0, 0)),
            pl.BlockSpec(bn1.shape, lambda i: (0, 0)),
            pl.BlockSpec(Wnd.shape, lambda i: (0, 0)),
            pl.BlockSpec(bnd.shape, lambda i: (0, 0)),
            pl.BlockSpec(Wg1.shape, lambda i: (0, 0)),
            pl.BlockSpec(bg1.shape, lambda i: (0, 0)),
            pl.BlockSpec(Wgd.shape, lambda i: (0, 0)),
            pl.BlockSpec(bgd.shape, lambda i: (0, 0)),
        ],
        out_specs=[
            pl.BlockSpec((blk, dn), lambda i: (i, 0)),
            pl.BlockSpec((1, 32), lambda i: (0, 0)),
        ],
        out_shape=[
            jax.ShapeDtypeStruct((n, dn), F32),
            jax.ShapeDtypeStruct((1, 32), F32),
        ],
        scratch_shapes=[
            pltpu.VMEM((1, dn), F32),
            pltpu.VMEM((1, 16), F32),
        ],
    )(x, acc, g, Wn1, bn1, Wnd, bnd, Wg1, bg1, Wgd, bgd)


def kernel(node_features, edge_index, edge_features, global_features,
           We1, be1, Wed, bed, Wn1, bn1, Wnd, bnd, Wg1, bg1, Wgd, bgd):
    x = node_features
    ef = edge_features
    g = global_features
    n = x.shape[0]
    e_num = ef.shape[0]
    src = edge_index[0]
    dst = edge_index[1]

    pq = _pq_call(x, We1, Wed)
    cef = _cef_call(ef, We1, Wed, be1.reshape(1, -1), bed.reshape(1, -1), g)
    zeros = jnp.zeros((n, 32), F32)
    ef_out, acc = _sc_edge_call(pq, cef, src, dst, zeros)
    nf_out, gf_out = _node_call(
        x, acc, g, Wn1, bn1.reshape(1, -1), Wnd, bnd.reshape(1, -1),
        Wg1, bg1.reshape(1, -1), Wgd, bgd.reshape(1, -1), e_num)
    return nf_out, ef_out, gf_out


# speedup vs baseline: 1.4701x; 1.4701x over previous
import functools

import jax
import jax.numpy as jnp
from jax import lax
from jax.experimental import pallas as pl
from jax.experimental.pallas import tpu as pltpu
from jax.experimental.pallas import tpu_sc as plsc

F32 = jnp.float32

NC = 2
NS = 16
NW = NC * NS


def _pq_body(x_ref, we1_ref, wed_ref, out_ref):
    wc = jnp.dot(we1_ref[0:256, :], wed_ref[...], preferred_element_type=F32)
    w = jnp.concatenate([wc[0:128, :], wc[128:256, :]], axis=1)
    out_ref[...] = jnp.dot(x_ref[...], w, preferred_element_type=F32)


def _pq_call(x, We1, Wed):
    n, dn = x.shape
    blk = 2000
    grid = n // blk
    return pl.pallas_call(
        _pq_body,
        grid=(grid,),
        in_specs=[
            pl.BlockSpec((blk, dn), lambda i: (i, 0)),
            pl.BlockSpec(We1.shape, lambda i: (0, 0)),
            pl.BlockSpec(Wed.shape, lambda i: (0, 0)),
        ],
        out_specs=pl.BlockSpec((blk, 32), lambda i: (i, 0)),
        out_shape=jax.ShapeDtypeStruct((n, 32), F32),
    )(x, We1, Wed)


def _cef_body(eft_ref, we1_ref, wed_ref, be1_ref, bed_ref, g_ref, out_ref):
    wed = wed_ref[...]
    wce = jnp.dot(we1_ref[256:272, :], wed, preferred_element_type=F32)
    wcg = jnp.dot(we1_ref[272:304, :], wed, preferred_element_type=F32)
    ct = (jnp.dot(g_ref[...], wcg, preferred_element_type=F32)
          + jnp.dot(be1_ref[...], wed, preferred_element_type=F32)
          + bed_ref[...])
    de = eft_ref.shape[0]
    eye = jnp.float32(1.0) * (lax.broadcasted_iota(jnp.int32, (de, de), 0)
                              == lax.broadcasted_iota(jnp.int32, (de, de), 1))
    wcat = jnp.concatenate([wce, eye], axis=1)
    ctcat = jnp.concatenate([ct, jnp.zeros_like(ct)], axis=1)
    out_ref[...] = lax.dot_general(
        eft_ref[...], wcat, (((0,), (0,)), ((), ())),
        preferred_element_type=F32) + ctcat


def _cef_call(eft, We1, Wed, be1, bed, g):
    de, e_num = eft.shape
    blk = 16000
    grid = e_num // blk
    return pl.pallas_call(
        _cef_body,
        grid=(grid,),
        in_specs=[
            pl.BlockSpec((de, blk), lambda i: (0, i)),
            pl.BlockSpec(We1.shape, lambda i: (0, 0)),
            pl.BlockSpec(Wed.shape, lambda i: (0, 0)),
            pl.BlockSpec(be1.shape, lambda i: (0, 0)),
            pl.BlockSpec(bed.shape, lambda i: (0, 0)),
            pl.BlockSpec(g.shape, lambda i: (0, 0)),
        ],
        out_specs=pl.BlockSpec((blk, 2 * de), lambda i: (i, 0)),
        out_shape=jax.ShapeDtypeStruct((e_num, 2 * de), F32),
    )(eft, We1, Wed, be1, bed, g)


def _sc_edge_body(n, e_num, k, nr, pq_hbm, cef_hbm, src_hbm, dst_hbm, z_hbm,
                  efo_hbm, acc_hbm,
                  idx_s, idx_d, sidx_s, sidx_d, gsrc, gdst, cefb,
                  payf, payb, eoutb, acc,
                  s_is, s_id, s_gs, s_gd, s_ce, s_eo, s_pf, s_pb):
    cid = lax.axis_index("c")
    sid = lax.axis_index("s")
    wid = sid * NC + cid
    nchunks = e_num // k
    nch = jnp.int32(nchunks // NW) + (wid < jnp.int32(nchunks % NW))
    stripe = 1000
    n_stripes = n // stripe

    @pl.when(sid < n_stripes)
    def _():
        pltpu.sync_copy(z_hbm.at[pl.ds(sid * stripe, stripe)],
                        acc.at[pl.ds(sid * stripe, stripe)])

    cnt_row = jnp.where(lax.iota(jnp.int32, 16) == 0,
                        jnp.float32(1.0), jnp.float32(0.0))
    for b in range(2):
        for e in range(k):
            payf[b][e, 16:32] = cnt_row
            payb[b][e, 16:32] = cnt_row

    plsc.subcore_barrier()

    def off(r):
        return (r * NW + wid) * k

    def issue_idx(r, b):
        @pl.when(r < nch)
        def _():
            pltpu.async_copy(src_hbm.at[pl.ds(off(r), k)], idx_s[b], s_is[b])
            pltpu.async_copy(dst_hbm.at[pl.ds(off(r), k)], idx_d[b], s_id[b])

    def wait_idx(r, b):
        @pl.when(r < nch)
        def _():
            pltpu.make_async_copy(src_hbm.at[pl.ds(0, k)], idx_s[b],
                                  s_is[b]).wait()
            pltpu.make_async_copy(dst_hbm.at[pl.ds(0, k)], idx_d[b],
                                  s_id[b]).wait()

    def issue_gather(r, b):
        @pl.when(r < nch)
        def _():
            pltpu.async_copy(pq_hbm.at[idx_s[b]], gsrc[b], s_gs[b])
            pltpu.async_copy(pq_hbm.at[idx_d[b]], gdst[b], s_gd[b])
            pltpu.async_copy(cef_hbm.at[pl.ds(off(r), k)], cefb[b], s_ce[b])

    def wait_gather(r, b):
        @pl.when(r < nch)
        def _():
            pltpu.make_async_copy(pq_hbm.at[idx_s[b]], gsrc[b], s_gs[b]).wait()
            pltpu.make_async_copy(pq_hbm.at[idx_d[b]], gdst[b], s_gd[b]).wait()
            pltpu.make_async_copy(cef_hbm.at[pl.ds(0, k)], cefb[b],
                                  s_ce[b]).wait()

    def issue_store(r, b):
        @pl.when(r < nch)
        def _():
            pltpu.async_copy(eoutb[b], efo_hbm.at[pl.ds(off(r), k)], s_eo[b])
            pltpu.async_copy(payf[b], acc.at[sidx_d[b]], s_pf[b], add=True)
            pltpu.async_copy(payb[b], acc.at[sidx_s[b]], s_pb[b], add=True)

    def wait_store(r, b):
        @pl.when((r >= 0) & (r < nch))
        def _():
            pltpu.make_async_copy(eoutb[b], efo_hbm.at[pl.ds(0, k)],
                                  s_eo[b]).wait()
            pltpu.make_async_copy(payf[b], acc.at[sidx_d[b]], s_pf[b]).wait()
            pltpu.make_async_copy(payb[b], acc.at[sidx_s[b]], s_pb[b]).wait()

    issue_idx(0, 0)
    issue_idx(1, 1)
    wait_idx(0, 0)
    issue_gather(0, 0)

    @pl.loop(0, nr, step=2)
    def _round_pair(i):
        for db in range(2):
            j = i + db
            b = db
            wait_store(j - 2, b)
            wait_gather(j, b)

            @pl.when(j < nch)
            def _():
                for t in range(k // 16):
                    sidx_s[b][pl.ds(t * 16, 16)] = idx_s[b][pl.ds(t * 16, 16)]
                    sidx_d[b][pl.ds(t * 16, 16)] = idx_d[b][pl.ds(t * 16, 16)]

            wait_idx(j + 1, 1 - b)
            issue_gather(j + 1, 1 - b)
            issue_idx(j + 2, b)

            @pl.when(j < nch)
            def _():
                for e in range(k):
                    ps = gsrc[b][e, 0:16]
                    qs = gsrc[b][e, 16:32]
                    pd = gdst[b][e, 0:16]
                    qd = gdst[b][e, 16:32]
                    ce = cefb[b][e, 0:16]
                    efv = cefb[b][e, 16:32]
                    fwd = ps + qd + ce
                    bwd = pd + qs + ce
                    payf[b][e, 0:16] = fwd
                    payb[b][e, 0:16] = bwd
                    eoutb[b][e, 0:16] = (fwd + bwd) * jnp.float32(0.5) + efv

            issue_store(j, b)

    wait_store(nr - 2, 0)
    wait_store(nr - 1, 1)

    plsc.subcore_barrier()

    @pl.when(sid < n_stripes)
    def _():
        pltpu.sync_copy(acc.at[pl.ds(sid * stripe, stripe)],
                        acc_hbm.at[cid].at[pl.ds(sid * stripe, stripe)])


def _sc_edge_call(pq, cef, src, dst, zeros):
    n = pq.shape[0]
    e_num = src.shape[0]
    k = 128
    nr = -(-(e_num // k) // NW)
    nr += nr % 2
    mesh = plsc.VectorSubcoreMesh(core_axis_name="c", subcore_axis_name="s")
    f = pl.kernel(
        functools.partial(_sc_edge_body, n, e_num, k, nr),
        out_type=[
            jax.ShapeDtypeStruct((e_num, 16), F32),
            jax.ShapeDtypeStruct((NC, n, 32), F32),
        ],
        mesh=mesh,
        scratch_types=[
            [pltpu.VMEM((k,), jnp.int32) for _ in range(2)],
            [pltpu.VMEM((k,), jnp.int32) for _ in range(2)],
            [pltpu.VMEM((k,), jnp.int32) for _ in range(2)],
            [pltpu.VMEM((k,), jnp.int32) for _ in range(2)],
            [pltpu.VMEM((k, 32), F32) for _ in range(2)],
            [pltpu.VMEM((k, 32), F32) for _ in range(2)],
            [pltpu.VMEM((k, 32), F32) for _ in range(2)],
            [pltpu.VMEM((k, 32), F32) for _ in range(2)],
            [pltpu.VMEM((k, 32), F32) for _ in range(2)],
            [pltpu.VMEM((k, 16), F32) for _ in range(2)],
            pltpu.VMEM_SHARED((n, 32), F32),
            [pltpu.SemaphoreType.DMA for _ in range(2)],
            [pltpu.SemaphoreType.DMA for _ in range(2)],
            [pltpu.SemaphoreType.DMA for _ in range(2)],
            [pltpu.SemaphoreType.DMA for _ in range(2)],
            [pltpu.SemaphoreType.DMA for _ in range(2)],
            [pltpu.SemaphoreType.DMA for _ in range(2)],
            [pltpu.SemaphoreType.DMA for _ in range(2)],
            [pltpu.SemaphoreType.DMA for _ in range(2)],
        ],
        compiler_params=pltpu.CompilerParams(use_tc_tiling_on_sc=False),
    )
    return f(pq, cef, src, dst, zeros)


def _node_body(nblk, n, e_num, x_ref, acc_ref, g_ref, wn1_ref, bn1_ref,
               wnd_ref, bnd_ref, wg1_ref, bg1_ref, wgd_ref, bgd_ref,
               nf_ref, gf_ref, snf_ref, seg_ref):
    i = pl.program_id(0)
    acc0 = acc_ref[0]
    acc1 = acc_ref[1]
    data = acc0[:, 0:16] + acc1[:, 0:16]
    cnt = acc0[:, 16:17] + acc1[:, 16:17]
    em = data / jnp.maximum(cnt, 1.0)

    wnd = wnd_ref[...]
    wn = jnp.dot(wn1_ref[...], wnd, preferred_element_type=F32)
    gterm = (jnp.dot(g_ref[...], wn[144:176, :], preferred_element_type=F32)
             + jnp.dot(bn1_ref[...], wnd, preferred_element_type=F32)
             + bnd_ref[...])
    xb = x_ref[...]
    nf_new = (jnp.dot(xb, wn[0:128, :], preferred_element_type=F32)
              + jnp.dot(em, wn[128:144, :], preferred_element_type=F32)
              + gterm)
    nf_ref[...] = nf_new + xb

    part_nf = jnp.sum(nf_new, axis=0, keepdims=True)
    part_eg = jnp.sum(data, axis=0, keepdims=True)

    @pl.when(i == 0)
    def _():
        snf_ref[...] = part_nf
        seg_ref[...] = part_eg

    @pl.when(i > 0)
    def _():
        snf_ref[...] += part_nf
        seg_ref[...] += part_eg

    @pl.when(i == nblk - 1)
    def _():
        ng = snf_ref[...] * jnp.float32(1.0 / n)
        eg = seg_ref[...] * jnp.float32(1.0 / (2.0 * e_num))
        wgd = wgd_ref[...]
        wg = jnp.dot(wg1_ref[...], wgd, preferred_element_type=F32)
        gf_new = (jnp.dot(eg, wg[0:16, :], preferred_element_type=F32)
                  + jnp.dot(ng, wg[16:144, :], preferred_element_type=F32)
                  + jnp.dot(g_ref[...], wg[144:176, :],
                            preferred_element_type=F32)
                  + jnp.dot(bg1_ref[...], wgd, preferred_element_type=F32)
                  + bgd_ref[...])
        gf_ref[...] = gf_new + g_ref[...]


def _node_call(x, acc, g, Wn1, bn1, Wnd, bnd, Wg1, bg1, Wgd, bgd, e_num):
    n, dn = x.shape
    blk = 2000
    grid = n // blk
    return pl.pallas_call(
        functools.partial(_node_body, grid, n, e_num),
        grid=(grid,),
        in_specs=[
            pl.BlockSpec((blk, dn), lambda i: (i, 0)),
            pl.BlockSpec((2, blk, 32), lambda i: (0, i, 0)),
            pl.BlockSpec(g.shape, lambda i: (0, 0)),
            pl.BlockSpec(Wn1.shape, lambda i: (0, 0)),
            pl.BlockSpec(bn1.shape, lambda i: (0, 0)),
            pl.BlockSpec(Wnd.shape, lambda i: (0, 0)),
            pl.BlockSpec(bnd.shape, lambda i: (0, 0)),
            pl.BlockSpec(Wg1.shape, lambda i: (0, 0)),
            pl.BlockSpec(bg1.shape, lambda i: (0, 0)),
            pl.BlockSpec(Wgd.shape, lambda i: (0, 0)),
            pl.BlockSpec(bgd.shape, lambda i: (0, 0)),
        ],
        out_specs=[
            pl.BlockSpec((blk, dn), lambda i: (i, 0)),
            pl.BlockSpec((1, 32), lambda i: (0, 0)),
        ],
        out_shape=[
            jax.ShapeDtypeStruct((n, dn), F32),
            jax.ShapeDtypeStruct((1, 32), F32),
        ],
        scratch_shapes=[
            pltpu.VMEM((1, dn), F32),
            pltpu.VMEM((1, 16), F32),
        ],
    )(x, acc, g, Wn1, bn1, Wnd, bnd, Wg1, bg1, Wgd, bgd)


def kernel(node_features, edge_index, edge_features, global_features,
           We1, be1, Wed, bed, Wn1, bn1, Wnd, bnd, Wg1, bg1, Wgd, bgd):
    x = node_features
    ef = edge_features
    g = global_features
    n = x.shape[0]
    e_num = ef.shape[0]
    src = edge_index[0]
    dst = edge_index[1]

    pq = _pq_call(x, We1, Wed)
    cef = _cef_call(ef.T, We1, Wed, be1.reshape(1, -1), bed.reshape(1, -1), g)
    zeros = jnp.zeros((n, 32), F32)
    ef_out, acc = _sc_edge_call(pq, cef, src, dst, zeros)
    nf_out, gf_out = _node_call(
        x, acc, g, Wn1, bn1.reshape(1, -1), Wnd, bnd.reshape(1, -1),
        Wg1, bg1.reshape(1, -1), Wgd, bgd.reshape(1, -1), e_num)
    return nf_out, ef_out, gf_out
